# baseline (speedup 1.0000x reference)
import jax
import jax.numpy as jnp
from jax import lax
from jax.experimental import pallas as pl
from jax.experimental.pallas import tpu as pltpu

B = 1024
N = 100000
GAMMA = 2.0

ROW_BLK = 64
NUM_ROW_BLKS = B // ROW_BLK
LANES = 128
NCH = N // LANES
MAIN = NCH * LANES
TAIL = N - MAIN


def _pad_tail(v):
    return jnp.concatenate(
        [v, jnp.zeros((ROW_BLK, LANES - TAIL), jnp.float32)], axis=1
    )


def _loss_body(x_ref, t_ref, a_ref, out_ref, sacc, qacc, aacc):
    r = pl.program_id(0)
    rows = pl.ds(r * ROW_BLK, ROW_BLK)
    t = t_ref[rows, :]
    lane = lax.broadcasted_iota(jnp.int32, (ROW_BLK, LANES), 1)
    tail_x = x_ref[:, pl.ds(MAIN, TAIL)]

    def max_body(i, mx):
        return jnp.maximum(mx, x_ref[:, pl.ds(i * LANES, LANES)])

    mx = lax.fori_loop(
        0, NCH, max_body, jnp.full((ROW_BLK, LANES), -jnp.inf, jnp.float32)
    )
    m = jnp.maximum(
        jnp.max(mx, axis=1, keepdims=True),
        jnp.max(tail_x, axis=1, keepdims=True),
    )

    def sum_body(i, carry):
        s, q, a = carry
        xi = x_ref[:, pl.ds(i * LANES, LANES)]
        ai = a_ref[:, pl.ds(i * LANES, LANES)]
        e = jnp.exp(xi - m)
        mask = (lane + i * LANES) == t
        return (
            s + e,
            q + jnp.where(mask, e, 0.0),
            a + jnp.where(mask, ai, 0.0),
        )

    zero = jnp.zeros((ROW_BLK, LANES), jnp.float32)
    s, q, a = lax.fori_loop(0, NCH, sum_body, (zero, zero, zero))

    mask_t = (lane[:, :TAIL] + MAIN) == t
    e_t = jnp.exp(tail_x - m)
    s = s + _pad_tail(e_t)
    q = q + _pad_tail(jnp.where(mask_t, e_t, 0.0))
    a = a + _pad_tail(jnp.where(mask_t, a_ref[:, pl.ds(MAIN, TAIL)], 0.0))

    sacc[rows, :] = s
    qacc[rows, :] = q
    aacc[rows, :] = a

    @pl.when(r == NUM_ROW_BLKS - 1)
    def _finish():
        s_all = jnp.sum(sacc[...], axis=1, keepdims=True)
        q_all = jnp.sum(qacc[...], axis=1, keepdims=True)
        at = jnp.sum(aacc[...], axis=1, keepdims=True)
        log_p = jnp.log(q_all) - jnp.log(s_all)
        one_m_p = 1.0 - q_all / s_all
        row_loss = -at * one_m_p * one_m_p * log_p
        out_ref[...] = (jnp.sum(row_loss) / B).reshape(1, 1)


def kernel(inputs, targets, alpha):
    targets = targets.reshape(B, 1).astype(jnp.int32)
    alpha_row = alpha.reshape(1, N)
    loss = pl.pallas_call(
        _loss_body,
        grid=(NUM_ROW_BLKS,),
        in_specs=[
            pl.BlockSpec((ROW_BLK, N), lambda r: (r, 0)),
            pl.BlockSpec((B, 1), lambda r: (0, 0)),
            pl.BlockSpec((1, N), lambda r: (0, 0)),
        ],
        out_specs=pl.BlockSpec((1, 1), lambda r: (0, 0)),
        out_shape=jax.ShapeDtypeStruct((1, 1), jnp.float32),
        scratch_shapes=[
            pltpu.VMEM((B, LANES), jnp.float32),
            pltpu.VMEM((B, LANES), jnp.float32),
            pltpu.VMEM((B, LANES), jnp.float32),
        ],
    )(inputs, targets, alpha_row)
    return loss[0, 0]


# speedup vs baseline: 2.7447x; 2.7447x over previous
import functools

import jax
import jax.numpy as jnp
from jax import lax
from jax.experimental import pallas as pl
from jax.experimental.pallas import tpu as pltpu
from jax.experimental.pallas import tpu_sc as plsc

B = 1024
N = 100000
GAMMA = 2.0

ROW_BLK = 64
NUM_ROW_BLKS = B // ROW_BLK

_NC = 2
_NS = 16
_NW = _NC * _NS
_BPW = B // _NW


def _sc_alpha_gather(a_flat, targets):
    mesh = plsc.VectorSubcoreMesh(core_axis_name="c", subcore_axis_name="s")

    @functools.partial(
        pl.kernel,
        mesh=mesh,
        out_type=jax.ShapeDtypeStruct((B,), jnp.float32),
        scratch_types=[
            pltpu.VMEM((_BPW,), jnp.int32),
            pltpu.VMEM((_BPW,), jnp.float32),
            pltpu.SemaphoreType.DMA,
        ],
    )
    def k(a_hbm, t_hbm, at_hbm, tgt_v, at_v, sem):
        wid = lax.axis_index("s") * _NC + lax.axis_index("c")
        base = wid * _BPW
        pltpu.sync_copy(t_hbm.at[pl.ds(base, _BPW)], tgt_v)
        pltpu.async_copy(a_hbm.at[tgt_v], at_v, sem).wait()
        pltpu.sync_copy(at_v, at_hbm.at[pl.ds(base, _BPW)])

    return k(a_flat, targets)


def _g_body(x_ref, t_ref, g_ref):
    r = pl.program_id(0)
    x = x_ref[...]
    m = jnp.max(x, axis=1, keepdims=True)
    t = t_ref[pl.ds(r * ROW_BLK, ROW_BLK), :]
    cols = lax.broadcasted_iota(jnp.int32, (ROW_BLK, N), 1)
    mask = cols == t
    e = jnp.exp(x - m)
    s = jnp.sum(e, axis=1, keepdims=True)
    xt = jnp.sum(jnp.where(mask, x, 0.0), axis=1, keepdims=True)
    log_p = (xt - m) - jnp.log(s)
    one_m_p = 1.0 - jnp.exp(log_p)
    g_ref[pl.ds(r * ROW_BLK, ROW_BLK), :] = -one_m_p * one_m_p * log_p


def _combine_body(g_ref, at_ref, out_ref):
    out_ref[...] = (jnp.sum(g_ref[...] * at_ref[...]) / B).reshape(1, 1)


def kernel(inputs, targets, alpha):
    targets = targets.reshape(-1).astype(jnp.int32)
    at = _sc_alpha_gather(alpha.reshape(-1), targets)
    g = pl.pallas_call(
        _g_body,
        grid=(NUM_ROW_BLKS,),
        in_specs=[
            pl.BlockSpec((ROW_BLK, N), lambda r: (r, 0)),
            pl.BlockSpec((B, 1), lambda r: (0, 0)),
        ],
        out_specs=pl.BlockSpec((B, 1), lambda r: (0, 0)),
        out_shape=jax.ShapeDtypeStruct((B, 1), jnp.float32),
    )(inputs, targets.reshape(B, 1))
    loss = pl.pallas_call(
        _combine_body,
        out_shape=jax.ShapeDtypeStruct((1, 1), jnp.float32),
    )(g, at.reshape(B, 1))
    return loss[0, 0]
